# baseline (speedup 1.0000x reference)
import functools

import jax
import jax.numpy as jnp
from jax import lax
from jax.experimental import pallas as pl
from jax.experimental.pallas import tpu as pltpu
from jax.experimental.pallas import tpu_sc as plsc

VOCAB = 250002
D_MODEL = 768
NUM_CLASSES = 2


_ROWS = 4096
_VPAD = 253952


def _proj_body(emb_ref, w_ref, b_ref, p0_ref, p1_ref):
    acc = lax.dot_general(
        w_ref[...], emb_ref[...],
        dimension_numbers=(((1,), (1,)), ((), ())),
        preferred_element_type=jnp.float32,
    ) + b_ref[...]
    p0_ref[...] = acc[0]
    p1_ref[...] = acc[1]


def _project_table(emb_table, fc_w, fc_b):
    w_pad = jnp.zeros((8, D_MODEL), jnp.float32).at[:NUM_CLASSES].set(fc_w)
    b_pad = jnp.zeros((8, 1), jnp.float32).at[:NUM_CLASSES, 0].set(fc_b)
    nb = _VPAD // _ROWS
    return pl.pallas_call(
        _proj_body,
        grid=(nb,),
        in_specs=[
            pl.BlockSpec((_ROWS, D_MODEL), lambda i: (i, 0)),
            pl.BlockSpec((8, D_MODEL), lambda i: (0, 0)),
            pl.BlockSpec((8, 1), lambda i: (0, 0)),
        ],
        out_specs=[
            pl.BlockSpec((_ROWS,), lambda i: (i,)),
            pl.BlockSpec((_ROWS,), lambda i: (i,)),
        ],
        out_shape=[
            jax.ShapeDtypeStruct((_VPAD,), jnp.float32),
            jax.ShapeDtypeStruct((_VPAD,), jnp.float32),
        ],
    )(emb_table, w_pad, b_pad)


_NC, _NS = 2, 16
_NW = _NC * _NS


def _make_gather(b_per_w):
    mesh = plsc.VectorSubcoreMesh(core_axis_name="c", subcore_axis_name="s")

    @functools.partial(
        pl.kernel,
        mesh=mesh,
        out_type=[
            jax.ShapeDtypeStruct((_NW * b_per_w,), jnp.float32),
            jax.ShapeDtypeStruct((_NW * b_per_w,), jnp.float32),
        ],
        scratch_types=[
            pltpu.VMEM((b_per_w,), jnp.int32),
            pltpu.VMEM((b_per_w,), jnp.float32),
            pltpu.VMEM((b_per_w,), jnp.float32),
            pltpu.SemaphoreType.DMA,
            pltpu.SemaphoreType.DMA,
        ],
        compiler_params=pltpu.CompilerParams(use_tc_tiling_on_sc=False),
    )
    def gather_k(p0_hbm, p1_hbm, idx_hbm, out0_hbm, out1_hbm,
                 idx_v, rows0_v, rows1_v, sem0, sem1):
        wid = lax.axis_index("s") * _NC + lax.axis_index("c")
        base = wid * b_per_w
        pltpu.sync_copy(idx_hbm.at[pl.ds(base, b_per_w)], idx_v)
        c0 = pltpu.async_copy(p0_hbm.at[idx_v], rows0_v, sem0)
        c1 = pltpu.async_copy(p1_hbm.at[idx_v], rows1_v, sem1)
        c0.wait()
        c1.wait()
        pltpu.sync_copy(rows0_v, out0_hbm.at[pl.ds(base, b_per_w)])
        pltpu.sync_copy(rows1_v, out1_hbm.at[pl.ds(base, b_per_w)])

    return gather_k


def kernel(x, emb_table, fc_w, fc_b):
    B, L = x.shape
    n_tok = B * L
    b_per_w = n_tok // _NW
    p0, p1 = _project_table(emb_table, fc_w, fc_b)
    idx = x.astype(jnp.int32).reshape(n_tok)
    out0, out1 = _make_gather(b_per_w)(p0, p1, idx)
    return jnp.stack([out0, out1], axis=-1).reshape(B, L, NUM_CLASSES)
